# Initial kernel scaffold; baseline (speedup 1.0000x reference)
import functools

import jax
import jax.numpy as jnp
from jax import lax
from jax.experimental import pallas as pl
from jax.experimental.pallas import tpu as pltpu
from jax.experimental.pallas import tpu_sc as plsc

N = 10000
E = 320000
D = 128
H = 8
DH = 16

NC = 2
NS = 16
L = 16

EPT = E // (NC * NS)
CH = 80
NCHUNK = EPT // CH
RPT = N // NS

_GDN = lax.GatherDimensionNumbers(
    offset_dims=(), collapsed_slice_dims=(0,), start_index_map=(0,))


def _gather16(vec, idx):
  return lax.gather(vec, idx.reshape(L, 1), _GDN, (1,),
                    mode=lax.GatherScatterMode.PROMISE_IN_BOUNDS)


def _sc_edge_kernel(h_hbm, c_hbm, src_hbm, dst_hbm, z128_hbm, z16_hbm,
                    acc_out, den_out,
                    acc_sh, den_sh, srcb, dstb, a1b, a2b, wb, hb, sb, sem):
  c = lax.axis_index("c")
  s = lax.axis_index("s")
  tile = c * NS + s
  ebase = tile * EPT

  rot = jnp.bitwise_and(lax.iota(jnp.int32, L) + 8, 15)
  lane = lax.iota(jnp.int32, L)
  valid8 = lane < 8

  r0 = s * RPT
  pltpu.sync_copy(z128_hbm.at[pl.ds(r0, RPT)], acc_sh.at[pl.ds(r0, RPT)])
  pltpu.sync_copy(z16_hbm.at[pl.ds(r0, RPT)], den_sh.at[pl.ds(r0, RPT)])
  plsc.subcore_barrier()

  def chunk_body(i, carry):
    base = ebase + i * CH
    pltpu.sync_copy(src_hbm.at[pl.ds(base, CH)], srcb)
    pltpu.sync_copy(dst_hbm.at[pl.ds(base, CH)], dstb)
    pltpu.async_copy(c_hbm.at[srcb], a1b, sem).wait()
    pltpu.async_copy(c_hbm.at[dstb], a2b, sem).wait()
    pltpu.async_copy(h_hbm.at[srcb], hb, sem).wait()

    def w_body(t, carry2):
      g1 = a1b[t]
      g2 = _gather16(a2b[t], rot)
      e = g1 + g2
      e = jnp.where(e >= 0.0, e, 0.2 * e)
      w = jnp.exp(e)
      wb[t] = jnp.where(valid8, w, 0.0)
      return carry2

    lax.fori_loop(0, CH, w_body, 0)
    pltpu.sync_copy(wb, den_sh.at[dstb], add=True)

    def s_body(t, carry2):
      wrow = wb[t]
      for j in range(H):
        wj = _gather16(wrow, jnp.full((L,), j, jnp.int32))
        sb[t, pl.ds(j * L, L)] = hb[t, pl.ds(j * L, L)] * wj
      return carry2

    lax.fori_loop(0, CH, s_body, 0)
    pltpu.sync_copy(sb, acc_sh.at[dstb], add=True)
    return carry

  lax.fori_loop(0, NCHUNK, chunk_body, 0)
  plsc.subcore_barrier()

  pltpu.sync_copy(acc_sh.at[pl.ds(r0, RPT)],
                  acc_out.at[pl.ds(c * N + r0, RPT)])
  pltpu.sync_copy(den_sh.at[pl.ds(r0, RPT)],
                  den_out.at[pl.ds(c * N + r0, RPT)])


_sc_edge = pl.kernel(
    _sc_edge_kernel,
    out_type=[jax.ShapeDtypeStruct((NC * N, D), jnp.float32),
              jax.ShapeDtypeStruct((NC * N, L), jnp.float32)],
    mesh=plsc.VectorSubcoreMesh(core_axis_name="c", subcore_axis_name="s",
                                num_cores=NC, num_subcores=NS),
    scratch_types=[
        pltpu.VMEM_SHARED((N, D), jnp.float32),
        pltpu.VMEM_SHARED((N, L), jnp.float32),
        pltpu.VMEM((CH,), jnp.int32),
        pltpu.VMEM((CH,), jnp.int32),
        pltpu.VMEM((CH, L), jnp.float32),
        pltpu.VMEM((CH, L), jnp.float32),
        pltpu.VMEM((CH, L), jnp.float32),
        pltpu.VMEM((CH, D), jnp.float32),
        pltpu.VMEM((CH, D), jnp.float32),
        pltpu.SemaphoreType.DMA,
    ],
)


RB = 1000
GRID = N // RB


def _tc_first_k(x_ref, wcat_ref, h_ref, c_ref):
  r = jnp.dot(x_ref[...], wcat_ref[...], preferred_element_type=jnp.float32)
  h_ref[...] = r[:, :D]
  c_ref[...] = r[:, D:]


def _tc_mid_k(a0_ref, a1_ref, d0_ref, d1_ref, b_ref, wcat_ref, erep_ref,
              h_ref, c_ref):
  den = d0_ref[...] + d1_ref[...] + 1e-16
  dexp = jnp.dot(den, erep_ref[...], preferred_element_type=jnp.float32)
  xx = (a0_ref[...] + a1_ref[...]) / dexp + b_ref[...]
  xx = jnp.maximum(xx, 0.0)
  r = jnp.dot(xx, wcat_ref[...], preferred_element_type=jnp.float32)
  h_ref[...] = r[:, :D]
  c_ref[...] = r[:, D:]


def _tc_last_k(a0_ref, a1_ref, d0_ref, d1_ref, b_ref, wf_ref, bf_ref,
               erep_ref, y_ref):
  den = d0_ref[...] + d1_ref[...] + 1e-16
  dexp = jnp.dot(den, erep_ref[...], preferred_element_type=jnp.float32)
  xx = (a0_ref[...] + a1_ref[...]) / dexp + b_ref[...]
  xx = jnp.maximum(xx, 0.0)
  y_ref[...] = (jnp.dot(xx, wf_ref[...], preferred_element_type=jnp.float32)
                + bf_ref[...])


def _row_spec(width):
  return pl.BlockSpec((RB, width), lambda i: (i, 0))


def _full_spec(shape):
  return pl.BlockSpec(shape, lambda i: tuple(0 for _ in shape))


_tc_first = pl.pallas_call(
    _tc_first_k,
    grid=(GRID,),
    in_specs=[_row_spec(D), _full_spec((D, D + L))],
    out_specs=[_row_spec(D), _row_spec(L)],
    out_shape=[jax.ShapeDtypeStruct((N, D), jnp.float32),
               jax.ShapeDtypeStruct((N, L), jnp.float32)],
)

_tc_mid = pl.pallas_call(
    _tc_mid_k,
    grid=(GRID,),
    in_specs=[_row_spec(D), _row_spec(D), _row_spec(L), _row_spec(L),
              _full_spec((1, D)), _full_spec((D, D + L)),
              _full_spec((L, D))],
    out_specs=[_row_spec(D), _row_spec(L)],
    out_shape=[jax.ShapeDtypeStruct((N, D), jnp.float32),
               jax.ShapeDtypeStruct((N, L), jnp.float32)],
)

_tc_last = pl.pallas_call(
    _tc_last_k,
    grid=(GRID,),
    in_specs=[_row_spec(D), _row_spec(D), _row_spec(L), _row_spec(L),
              _full_spec((1, D)), _full_spec((D, D)), _full_spec((1, D)),
              _full_spec((L, D))],
    out_specs=_row_spec(D),
    out_shape=jax.ShapeDtypeStruct((N, D), jnp.float32),
)


def _mcat(a_s, a_d):
  eye = jnp.eye(H, dtype=jnp.float32)
  ms = (a_s[:, :, None] * eye[:, None, :]).reshape(D, H)
  md = (a_d[:, :, None] * eye[:, None, :]).reshape(D, H)
  return jnp.concatenate([ms, md], axis=1)


@jax.jit
def _run(x, src, dst, params):
  (W0, a_src0, a_dst0, b0, W1, a_src1, a_dst1, b1,
   W2, a_src2, a_dst2, b2, Wv, Wo, Wp, bv, bo, bp) = params

  wcats = [jnp.concatenate([W, _mcat(a_s, a_d)], axis=1)
           for (W, a_s, a_d) in ((W0, a_src0, a_dst0),
                                 (W1, a_src1, a_dst1),
                                 (W2, a_src2, a_dst2))]
  biases = [b0, b1, b2]
  wf = Wv @ Wo @ Wp
  bf = bv @ Wo @ Wp + bo @ Wp + bp

  erep_top = (jnp.eye(H, dtype=jnp.float32)[:, :, None]
              * jnp.ones((DH,), jnp.float32)).reshape(H, D)
  erep = jnp.concatenate([erep_top, jnp.zeros((H, D), jnp.float32)], axis=0)

  z128 = jnp.zeros((N, D), jnp.float32)
  z16 = jnp.zeros((N, L), jnp.float32)

  h, cc = _tc_first(x, wcats[0])
  for l in range(3):
    accf, denf = _sc_edge(h, cc, src, dst, z128, z16)
    a0, a1 = accf[:N], accf[N:]
    d0, d1 = denf[:N], denf[N:]
    if l < 2:
      h, cc = _tc_mid(a0, a1, d0, d1, biases[l][None, :], wcats[l + 1], erep)
    else:
      y = _tc_last(a0, a1, d0, d1, biases[l][None, :], wf, bf[None, :], erep)
  return y


def kernel(x, edge_index, W0, a_src0, a_dst0, b0, W1, a_src1, a_dst1, b1,
           W2, a_src2, a_dst2, b2, Wq, Wk, Wv, Wo, Wp, bq, bk, bv, bo, bp):
  src = edge_index[0]
  dst = edge_index[1]
  params = (W0, a_src0, a_dst0, b0, W1, a_src1, a_dst1, b1,
            W2, a_src2, a_dst2, b2, Wv, Wo, Wp, bv, bo, bp)
  return _run(x, src, dst, params)


# speedup vs baseline: 48.3576x; 48.3576x over previous
import functools

import jax
import jax.numpy as jnp
from jax import lax
from jax.experimental import pallas as pl
from jax.experimental.pallas import tpu as pltpu
from jax.experimental.pallas import tpu_sc as plsc

N = 10000
E = 320000
D = 128
H = 8
DH = 16

NC = 2
NS = 16
L = 16

EPT = E // (NC * NS)
CH = 80
NCHUNK = EPT // CH
ST = 624
NREM = N - NS * ST
ND = 1256
STD = 80
NREMD = ND - (NS - 1) * STD

_GDN = lax.GatherDimensionNumbers(
    offset_dims=(), collapsed_slice_dims=(0,), start_index_map=(0,))


def _gather16(vec, idx):
  return lax.gather(vec, idx.reshape(L, 1), _GDN, (1,),
                    mode=lax.GatherScatterMode.PROMISE_IN_BOUNDS)


def _sc_edge_kernel(h_hbm, c_hbm, src_hbm, dst_hbm, z128_hbm,
                    acc_out, den_out,
                    acc_sh, den_sh, srcb, dstb, dib, buf1, buf2, wb,
                    sem):
  c = lax.axis_index("c")
  s = lax.axis_index("s")
  tile = c * NS + s
  ebase = tile * EPT

  rot = jnp.bitwise_and(lax.iota(jnp.int32, L) + 8, 15)
  lane = lax.iota(jnp.int32, L)
  valid8 = lane < 8
  zrow = jnp.zeros((L,), jnp.float32)

  r0 = s * ST
  pltpu.sync_copy(z128_hbm.at[pl.ds(r0, ST)], acc_sh.at[pl.ds(r0, ST)])

  @pl.when(s == NS - 1)
  def _zero_rem():
    pltpu.sync_copy(z128_hbm.at[pl.ds(NS * ST, NREM)],
                    acc_sh.at[pl.ds(NS * ST, NREM)])
    pltpu.sync_copy(z128_hbm.at[pl.ds(0, NREMD)],
                    den_sh.at[pl.ds((NS - 1) * STD, NREMD)])

  @pl.when(s < NS - 1)
  def _zero_den():
    pltpu.sync_copy(z128_hbm.at[pl.ds(0, STD)],
                    den_sh.at[pl.ds(s * STD, STD)])

  plsc.subcore_barrier()

  def chunk_body(i, carry):
    base = ebase + i * CH
    pltpu.sync_copy(src_hbm.at[pl.ds(base, CH)], srcb)
    pltpu.sync_copy(dst_hbm.at[pl.ds(base, CH)], dstb)
    pltpu.async_copy(c_hbm.at[srcb], buf1, sem).wait()
    pltpu.async_copy(c_hbm.at[dstb], buf2, sem).wait()

    def w_body(tb, carry2):
      dv = dstb[pl.ds(tb * L, L)]
      dib[pl.ds(tb * L, L)] = lax.shift_right_logical(dv, 3)
      for k in range(L):
        t = tb * L + k
        g1 = buf1[t, pl.ds(0, L)]
        g2 = _gather16(buf2[t, pl.ds(0, L)], rot)
        e = g1 + g2
        e = jnp.where(e >= 0.0, e, 0.2 * e)
        w = jnp.exp(e)
        wm = jnp.where(valid8, w, 0.0)
        wb[t] = wm
        db = _gather16(dv, jnp.full((L,), k, jnp.int32))
        mf = jnp.bitwise_and(db, 7).astype(jnp.float32)
        for j in range(8):
          dj = mf - float(j)
          buf2[t, pl.ds(j * L, L)] = wm * jnp.maximum(1.0 - dj * dj, 0.0)
      return carry2

    lax.fori_loop(0, CH // L, w_body, 0)
    pltpu.sync_copy(buf2, den_sh.at[dib], add=True)

    pltpu.async_copy(h_hbm.at[srcb], buf1, sem).wait()

    def s_body(t, carry2):
      wrow = wb[t]
      for j in range(H):
        wj = _gather16(wrow, jnp.full((L,), j, jnp.int32))
        buf1[t, pl.ds(j * L, L)] = buf1[t, pl.ds(j * L, L)] * wj
      return carry2

    lax.fori_loop(0, CH, s_body, 0)
    pltpu.sync_copy(buf1, acc_sh.at[dstb], add=True)
    return carry

  lax.fori_loop(0, NCHUNK, chunk_body, 0)
  plsc.subcore_barrier()

  pltpu.sync_copy(acc_sh.at[pl.ds(r0, ST)],
                  acc_out.at[pl.ds(c * N + r0, ST)])

  @pl.when(s < NS - 1)
  def _export_den():
    pltpu.sync_copy(den_sh.at[pl.ds(s * STD, STD)],
                    den_out.at[pl.ds(c * ND + s * STD, STD)])

  @pl.when(s == NS - 1)
  def _export_rem():
    pltpu.sync_copy(acc_sh.at[pl.ds(NS * ST, NREM)],
                    acc_out.at[pl.ds(c * N + NS * ST, NREM)])
    pltpu.sync_copy(den_sh.at[pl.ds((NS - 1) * STD, NREMD)],
                    den_out.at[pl.ds(c * ND + (NS - 1) * STD, NREMD)])


_sc_edge = pl.kernel(
    _sc_edge_kernel,
    out_type=[jax.ShapeDtypeStruct((NC * N, D), jnp.float32),
              jax.ShapeDtypeStruct((NC * ND, D), jnp.float32)],
    mesh=plsc.VectorSubcoreMesh(core_axis_name="c", subcore_axis_name="s",
                                num_cores=NC, num_subcores=NS),
    scratch_types=[
        pltpu.VMEM_SHARED((N, D), jnp.float32),
        pltpu.VMEM_SHARED((ND, D), jnp.float32),
        pltpu.VMEM((CH,), jnp.int32),
        pltpu.VMEM((CH,), jnp.int32),
        pltpu.VMEM((CH,), jnp.int32),
        pltpu.VMEM((CH, D), jnp.float32),
        pltpu.VMEM((CH, D), jnp.float32),
        pltpu.VMEM((CH, L), jnp.float32),
        pltpu.SemaphoreType.DMA,
    ],
)


RB = 1000
GRID = N // RB


def _tc_first_k(x_ref, wcat_ref, h_ref, c_ref):
  r = jnp.dot(x_ref[...], wcat_ref[...], preferred_element_type=jnp.float32)
  h_ref[...] = r[:, :D]
  c_ref[...] = r[:, D:]


def _tc_mid_k(a0_ref, a1_ref, den_ref, b_ref, wcat_ref, erep_ref,
              h_ref, c_ref):
  den = den_ref[...] + 1e-16
  dexp = jnp.dot(den, erep_ref[...], preferred_element_type=jnp.float32)
  xx = (a0_ref[...] + a1_ref[...]) / dexp + b_ref[...]
  xx = jnp.maximum(xx, 0.0)
  r = jnp.dot(xx, wcat_ref[...], preferred_element_type=jnp.float32)
  h_ref[...] = r[:, :D]
  c_ref[...] = r[:, D:]


def _tc_last_k(a0_ref, a1_ref, den_ref, b_ref, wf_ref, bf_ref,
               erep_ref, y_ref):
  den = den_ref[...] + 1e-16
  dexp = jnp.dot(den, erep_ref[...], preferred_element_type=jnp.float32)
  xx = (a0_ref[...] + a1_ref[...]) / dexp + b_ref[...]
  xx = jnp.maximum(xx, 0.0)
  y_ref[...] = (jnp.dot(xx, wf_ref[...], preferred_element_type=jnp.float32)
                + bf_ref[...])


def _row_spec(width):
  return pl.BlockSpec((RB, width), lambda i: (i, 0))


def _full_spec(shape):
  return pl.BlockSpec(shape, lambda i: tuple(0 for _ in shape))


_tc_first = pl.pallas_call(
    _tc_first_k,
    grid=(GRID,),
    in_specs=[_row_spec(D), _full_spec((D, 2 * D))],
    out_specs=[_row_spec(D), _row_spec(D)],
    out_shape=[jax.ShapeDtypeStruct((N, D), jnp.float32),
               jax.ShapeDtypeStruct((N, D), jnp.float32)],
)

_tc_mid = pl.pallas_call(
    _tc_mid_k,
    grid=(GRID,),
    in_specs=[_row_spec(D), _row_spec(D), _row_spec(L),
              _full_spec((1, D)), _full_spec((D, 2 * D)),
              _full_spec((L, D))],
    out_specs=[_row_spec(D), _row_spec(D)],
    out_shape=[jax.ShapeDtypeStruct((N, D), jnp.float32),
               jax.ShapeDtypeStruct((N, D), jnp.float32)],
)

_tc_last = pl.pallas_call(
    _tc_last_k,
    grid=(GRID,),
    in_specs=[_row_spec(D), _row_spec(D), _row_spec(L),
              _full_spec((1, D)), _full_spec((D, D)), _full_spec((1, D)),
              _full_spec((L, D))],
    out_specs=_row_spec(D),
    out_shape=jax.ShapeDtypeStruct((N, D), jnp.float32),
)


def _mcat(a_s, a_d):
  eye = jnp.eye(H, dtype=jnp.float32)
  ms = (a_s[:, :, None] * eye[:, None, :]).reshape(D, H)
  md = (a_d[:, :, None] * eye[:, None, :]).reshape(D, H)
  return jnp.concatenate(
      [ms, md, jnp.zeros((D, D - 2 * H), jnp.float32)], axis=1)


@jax.jit
def _run(x, src, dst, params):
  (W0, a_src0, a_dst0, b0, W1, a_src1, a_dst1, b1,
   W2, a_src2, a_dst2, b2, Wv, Wo, Wp, bv, bo, bp) = params

  wcats = [jnp.concatenate([W, W @ _mcat(a_s, a_d)], axis=1)
           for (W, a_s, a_d) in ((W0, a_src0, a_dst0),
                                 (W1, a_src1, a_dst1),
                                 (W2, a_src2, a_dst2))]
  biases = [b0, b1, b2]
  wf = Wv @ Wo @ Wp
  bf = bv @ Wo @ Wp + bo @ Wp + bp

  erep_top = (jnp.eye(H, dtype=jnp.float32)[:, :, None]
              * jnp.ones((DH,), jnp.float32)).reshape(H, D)
  erep = jnp.concatenate([erep_top, jnp.zeros((H, D), jnp.float32)], axis=0)

  z128 = jnp.zeros((N, D), jnp.float32)

  h, cc = _tc_first(x, wcats[0])
  for l in range(3):
    accf, denf = _sc_edge(h, cc, src, dst, z128)
    a0, a1 = accf[:N], accf[N:]
    den = (denf[:ND] + denf[ND:])[:N // 8].reshape(N, L)
    if l < 2:
      h, cc = _tc_mid(a0, a1, den, biases[l][None, :], wcats[l + 1], erep)
    else:
      y = _tc_last(a0, a1, den, biases[l][None, :], wf, bf[None, :], erep)
  return y


def kernel(x, edge_index, W0, a_src0, a_dst0, b0, W1, a_src1, a_dst1, b1,
           W2, a_src2, a_dst2, b2, Wq, Wk, Wv, Wo, Wp, bq, bk, bv, bo, bp):
  src = edge_index[0]
  dst = edge_index[1]
  params = (W0, a_src0, a_dst0, b0, W1, a_src1, a_dst1, b1,
            W2, a_src2, a_dst2, b2, Wv, Wo, Wp, bv, bo, bp)
  return _run(x, src, dst, params)


# speedup vs baseline: 58.2358x; 1.2043x over previous
import functools

import jax
import jax.numpy as jnp
from jax import lax
from jax.experimental import pallas as pl
from jax.experimental.pallas import tpu as pltpu
from jax.experimental.pallas import tpu_sc as plsc

N = 10000
E = 320000
D = 128
H = 8
DH = 16

NC = 2
NS = 16
L = 16

EPT = E // (NC * NS)
CH = 80
NCHUNK = EPT // CH
ST = 624
NREM = N - NS * ST
ND = 1256
STD = 80
NREMD = ND - (NS - 1) * STD

_GDN = lax.GatherDimensionNumbers(
    offset_dims=(), collapsed_slice_dims=(0,), start_index_map=(0,))


def _gather16(vec, idx):
  return lax.gather(vec, idx.reshape(L, 1), _GDN, (1,),
                    mode=lax.GatherScatterMode.PROMISE_IN_BOUNDS)


def _sc_edge_kernel(h_hbm, c_hbm, src_hbm, dst_hbm, z128_hbm,
                    acc_out, den_out,
                    acc_sh, den_sh, srcb, dstb, dib, buf1, buf2, wb,
                    sem1, sem2):
  c = lax.axis_index("c")
  s = lax.axis_index("s")
  tile = c * NS + s
  ebase = tile * EPT

  rot = jnp.bitwise_and(lax.iota(jnp.int32, L) + 8, 15)
  lane = lax.iota(jnp.int32, L)
  valid8 = lane < 8
  zrow = jnp.zeros((L,), jnp.float32)

  r0 = s * ST
  pltpu.sync_copy(z128_hbm.at[pl.ds(r0, ST)], acc_sh.at[pl.ds(r0, ST)])

  @pl.when(s == NS - 1)
  def _zero_rem():
    pltpu.sync_copy(z128_hbm.at[pl.ds(NS * ST, NREM)],
                    acc_sh.at[pl.ds(NS * ST, NREM)])
    pltpu.sync_copy(z128_hbm.at[pl.ds(0, NREMD)],
                    den_sh.at[pl.ds((NS - 1) * STD, NREMD)])

  @pl.when(s < NS - 1)
  def _zero_den():
    pltpu.sync_copy(z128_hbm.at[pl.ds(0, STD)],
                    den_sh.at[pl.ds(s * STD, STD)])

  plsc.subcore_barrier()

  def chunk_body(i, carry):
    base = ebase + i * CH
    pltpu.sync_copy(src_hbm.at[pl.ds(base, CH)], srcb)
    pltpu.sync_copy(dst_hbm.at[pl.ds(base, CH)], dstb)
    g_cs = pltpu.async_copy(c_hbm.at[srcb], buf1, sem1)
    g_cd = pltpu.async_copy(c_hbm.at[dstb], buf2, sem2)
    g_cs.wait()
    g_cd.wait()

    def w_body(tb, carry2):
      dv = dstb[pl.ds(tb * L, L)]
      dib[pl.ds(tb * L, L)] = lax.shift_right_logical(dv, 3)
      for k in range(L):
        t = tb * L + k
        g1 = buf1[t, pl.ds(0, L)]
        g2 = _gather16(buf2[t, pl.ds(0, L)], rot)
        e = g1 + g2
        e = jnp.where(e >= 0.0, e, 0.2 * e)
        w = jnp.exp(e)
        wm = jnp.where(valid8, w, 0.0)
        wb[t] = wm
        db = _gather16(dv, jnp.full((L,), k, jnp.int32))
        mf = jnp.bitwise_and(db, 7).astype(jnp.float32)
        for j in range(8):
          dj = mf - float(j)
          buf2[t, pl.ds(j * L, L)] = wm * jnp.maximum(1.0 - dj * dj, 0.0)
      return carry2

    lax.fori_loop(0, CH // L, w_body, 0)
    g_h = pltpu.async_copy(h_hbm.at[srcb], buf1, sem1)
    pltpu.sync_copy(buf2, den_sh.at[dib], add=True)
    g_h.wait()

    def s_body(t, carry2):
      wrow = wb[t]
      for j in range(H):
        wj = _gather16(wrow, jnp.full((L,), j, jnp.int32))
        buf1[t, pl.ds(j * L, L)] = buf1[t, pl.ds(j * L, L)] * wj
      return carry2

    lax.fori_loop(0, CH, s_body, 0)
    pltpu.sync_copy(buf1, acc_sh.at[dstb], add=True)
    return carry

  lax.fori_loop(0, NCHUNK, chunk_body, 0)
  plsc.subcore_barrier()

  pltpu.sync_copy(acc_sh.at[pl.ds(r0, ST)],
                  acc_out.at[pl.ds(c * N + r0, ST)])

  @pl.when(s < NS - 1)
  def _export_den():
    pltpu.sync_copy(den_sh.at[pl.ds(s * STD, STD)],
                    den_out.at[pl.ds(c * ND + s * STD, STD)])

  @pl.when(s == NS - 1)
  def _export_rem():
    pltpu.sync_copy(acc_sh.at[pl.ds(NS * ST, NREM)],
                    acc_out.at[pl.ds(c * N + NS * ST, NREM)])
    pltpu.sync_copy(den_sh.at[pl.ds((NS - 1) * STD, NREMD)],
                    den_out.at[pl.ds(c * ND + (NS - 1) * STD, NREMD)])


_sc_edge = pl.kernel(
    _sc_edge_kernel,
    out_type=[jax.ShapeDtypeStruct((NC * N, D), jnp.float32),
              jax.ShapeDtypeStruct((NC * ND, D), jnp.float32)],
    mesh=plsc.VectorSubcoreMesh(core_axis_name="c", subcore_axis_name="s",
                                num_cores=NC, num_subcores=NS),
    scratch_types=[
        pltpu.VMEM_SHARED((N, D), jnp.float32),
        pltpu.VMEM_SHARED((ND, D), jnp.float32),
        pltpu.VMEM((CH,), jnp.int32),
        pltpu.VMEM((CH,), jnp.int32),
        pltpu.VMEM((CH,), jnp.int32),
        pltpu.VMEM((CH, D), jnp.float32),
        pltpu.VMEM((CH, D), jnp.float32),
        pltpu.VMEM((CH, L), jnp.float32),
        pltpu.SemaphoreType.DMA,
        pltpu.SemaphoreType.DMA,
    ],
)


RB = 1000
GRID = N // RB


def _tc_first_k(x_ref, wcat_ref, h_ref, c_ref):
  r = jnp.dot(x_ref[...], wcat_ref[...], preferred_element_type=jnp.float32)
  h_ref[...] = r[:, :D]
  c_ref[...] = r[:, D:]


def _tc_mid_k(a0_ref, a1_ref, den_ref, b_ref, wcat_ref, erep_ref,
              h_ref, c_ref):
  den = den_ref[...] + 1e-16
  dexp = jnp.dot(den, erep_ref[...], preferred_element_type=jnp.float32)
  xx = (a0_ref[...] + a1_ref[...]) / dexp + b_ref[...]
  xx = jnp.maximum(xx, 0.0)
  r = jnp.dot(xx, wcat_ref[...], preferred_element_type=jnp.float32)
  h_ref[...] = r[:, :D]
  c_ref[...] = r[:, D:]


def _tc_last_k(a0_ref, a1_ref, den_ref, b_ref, wf_ref, bf_ref,
               erep_ref, y_ref):
  den = den_ref[...] + 1e-16
  dexp = jnp.dot(den, erep_ref[...], preferred_element_type=jnp.float32)
  xx = (a0_ref[...] + a1_ref[...]) / dexp + b_ref[...]
  xx = jnp.maximum(xx, 0.0)
  y_ref[...] = (jnp.dot(xx, wf_ref[...], preferred_element_type=jnp.float32)
                + bf_ref[...])


def _row_spec(width):
  return pl.BlockSpec((RB, width), lambda i: (i, 0))


def _full_spec(shape):
  return pl.BlockSpec(shape, lambda i: tuple(0 for _ in shape))


_tc_first = pl.pallas_call(
    _tc_first_k,
    grid=(GRID,),
    in_specs=[_row_spec(D), _full_spec((D, 2 * D))],
    out_specs=[_row_spec(D), _row_spec(D)],
    out_shape=[jax.ShapeDtypeStruct((N, D), jnp.float32),
               jax.ShapeDtypeStruct((N, D), jnp.float32)],
)

_tc_mid = pl.pallas_call(
    _tc_mid_k,
    grid=(GRID,),
    in_specs=[_row_spec(D), _row_spec(D), _row_spec(L),
              _full_spec((1, D)), _full_spec((D, 2 * D)),
              _full_spec((L, D))],
    out_specs=[_row_spec(D), _row_spec(D)],
    out_shape=[jax.ShapeDtypeStruct((N, D), jnp.float32),
               jax.ShapeDtypeStruct((N, D), jnp.float32)],
)

_tc_last = pl.pallas_call(
    _tc_last_k,
    grid=(GRID,),
    in_specs=[_row_spec(D), _row_spec(D), _row_spec(L),
              _full_spec((1, D)), _full_spec((D, D)), _full_spec((1, D)),
              _full_spec((L, D))],
    out_specs=_row_spec(D),
    out_shape=jax.ShapeDtypeStruct((N, D), jnp.float32),
)


def _mcat(a_s, a_d):
  eye = jnp.eye(H, dtype=jnp.float32)
  ms = (a_s[:, :, None] * eye[:, None, :]).reshape(D, H)
  md = (a_d[:, :, None] * eye[:, None, :]).reshape(D, H)
  return jnp.concatenate(
      [ms, md, jnp.zeros((D, D - 2 * H), jnp.float32)], axis=1)


@jax.jit
def _run(x, src, dst, params):
  (W0, a_src0, a_dst0, b0, W1, a_src1, a_dst1, b1,
   W2, a_src2, a_dst2, b2, Wv, Wo, Wp, bv, bo, bp) = params

  wcats = [jnp.concatenate([W, W @ _mcat(a_s, a_d)], axis=1)
           for (W, a_s, a_d) in ((W0, a_src0, a_dst0),
                                 (W1, a_src1, a_dst1),
                                 (W2, a_src2, a_dst2))]
  biases = [b0, b1, b2]
  wf = Wv @ Wo @ Wp
  bf = bv @ Wo @ Wp + bo @ Wp + bp

  erep_top = (jnp.eye(H, dtype=jnp.float32)[:, :, None]
              * jnp.ones((DH,), jnp.float32)).reshape(H, D)
  erep = jnp.concatenate([erep_top, jnp.zeros((H, D), jnp.float32)], axis=0)

  z128 = jnp.zeros((N, D), jnp.float32)

  h, cc = _tc_first(x, wcats[0])
  for l in range(3):
    accf, denf = _sc_edge(h, cc, src, dst, z128)
    a0, a1 = accf[:N], accf[N:]
    den = (denf[:ND] + denf[ND:])[:N // 8].reshape(N, L)
    if l < 2:
      h, cc = _tc_mid(a0, a1, den, biases[l][None, :], wcats[l + 1], erep)
    else:
      y = _tc_last(a0, a1, den, biases[l][None, :], wf, bf[None, :], erep)
  return y


def kernel(x, edge_index, W0, a_src0, a_dst0, b0, W1, a_src1, a_dst1, b1,
           W2, a_src2, a_dst2, b2, Wq, Wk, Wv, Wo, Wp, bq, bk, bv, bo, bp):
  src = edge_index[0]
  dst = edge_index[1]
  params = (W0, a_src0, a_dst0, b0, W1, a_src1, a_dst1, b1,
            W2, a_src2, a_dst2, b2, Wv, Wo, Wp, bv, bo, bp)
  return _run(x, src, dst, params)
